# Initial kernel scaffold; baseline (speedup 1.0000x reference)
import jax
import jax.numpy as jnp
from jax.experimental import pallas as pl


def kernel(f_nuc, f_bond, node_graph, message_graph, scope, W_local, W_msg, W_node):
    raise NotImplementedError("write your pallas kernel here")


# speedup vs baseline: 2.8673x; 2.8673x over previous
import jax
import jax.numpy as jnp
from jax import lax
from jax.experimental import pallas as pl
from jax.experimental.pallas import tpu as pltpu
from jax.experimental.pallas import tpu_sc as plsc

_DEPTH = 5
_NC = 2
_NS = 16
_NW = _NC * _NS
_CHUNK = 128


def _gather_sum_sc(table, i0, i1, i2):
    rp = i0.shape[0]
    h = table.shape[1]
    n_chunks = rp // _CHUNK
    loops = (n_chunks + _NW - 1) // _NW
    mesh = plsc.VectorSubcoreMesh(core_axis_name="c", subcore_axis_name="s")

    def body(i0_hbm, i1_hbm, i2_hbm, table_hbm, out_hbm,
             i0v, i1v, i2v, r0, r1, r2, sem):
        wid = lax.axis_index("s") * _NC + lax.axis_index("c")

        def chunk_body(j, carry):
            chunk = wid + j * _NW

            @pl.when(chunk < n_chunks)
            def _():
                base = chunk * _CHUNK
                pltpu.sync_copy(i0_hbm.at[pl.ds(base, _CHUNK)], i0v)
                pltpu.sync_copy(i1_hbm.at[pl.ds(base, _CHUNK)], i1v)
                pltpu.sync_copy(i2_hbm.at[pl.ds(base, _CHUNK)], i2v)
                c0 = pltpu.async_copy(table_hbm.at[i0v], r0, sem)
                c1 = pltpu.async_copy(table_hbm.at[i1v], r1, sem)
                c2 = pltpu.async_copy(table_hbm.at[i2v], r2, sem)
                c0.wait()
                c1.wait()
                c2.wait()

                def row_body(r, acc):
                    for v in range(h // 16):
                        sl = pl.ds(v * 16, 16)
                        r0[r, sl] = r0[r, sl] + r1[r, sl] + r2[r, sl]
                    return acc

                lax.fori_loop(0, _CHUNK, row_body, 0)
                pltpu.sync_copy(r0, out_hbm.at[pl.ds(base, _CHUNK)])

            return carry

        lax.fori_loop(0, loops, chunk_body, 0)

    k = pl.kernel(
        body,
        out_type=jax.ShapeDtypeStruct((rp, h), jnp.float32),
        mesh=mesh,
        scratch_types=[
            pltpu.VMEM((_CHUNK,), jnp.int32),
            pltpu.VMEM((_CHUNK,), jnp.int32),
            pltpu.VMEM((_CHUNK,), jnp.int32),
            pltpu.VMEM((_CHUNK, h), jnp.float32),
            pltpu.VMEM((_CHUNK, h), jnp.float32),
            pltpu.VMEM((_CHUNK, h), jnp.float32),
            pltpu.SemaphoreType.DMA,
        ],
    )
    return k(i0, i1, i2, table)


def _dot_t(a, b):
    return lax.dot_general(a, b, (((1,), (1,)), ((), ())),
                           preferred_element_type=jnp.float32)


def _local_tc(f_bond_p, w_local, rb):
    mp, fd = f_bond_p.shape
    h = w_local.shape[0]

    def body(fb_ref, wl_ref, lp_ref, msg_ref):
        lp = _dot_t(fb_ref[...], wl_ref[...])
        lp_ref[...] = lp
        msg_ref[...] = jnp.maximum(lp, 0.0)

    return pl.pallas_call(
        body,
        grid=(mp // rb,),
        in_specs=[
            pl.BlockSpec((rb, fd), lambda i: (i, 0)),
            pl.BlockSpec((h, fd), lambda i: (0, 0)),
        ],
        out_specs=[
            pl.BlockSpec((rb, h), lambda i: (i, 0)),
            pl.BlockSpec((rb, h), lambda i: (i, 0)),
        ],
        out_shape=[
            jax.ShapeDtypeStruct((mp, h), jnp.float32),
            jax.ShapeDtypeStruct((mp, h), jnp.float32),
        ],
        compiler_params=pltpu.CompilerParams(
            dimension_semantics=("arbitrary",)),
    )(f_bond_p, w_local)


def _msg_update_tc(s, lp, w_msg, rb):
    mp, h = s.shape

    def body(s_ref, lp_ref, w_ref, out_ref):
        out_ref[...] = jnp.maximum(
            lp_ref[...] + _dot_t(s_ref[...], w_ref[...]), 0.0)

    return pl.pallas_call(
        body,
        grid=(mp // rb,),
        in_specs=[
            pl.BlockSpec((rb, h), lambda i: (i, 0)),
            pl.BlockSpec((rb, h), lambda i: (i, 0)),
            pl.BlockSpec((h, h), lambda i: (0, 0)),
        ],
        out_specs=pl.BlockSpec((rb, h), lambda i: (i, 0)),
        out_shape=jax.ShapeDtypeStruct((mp, h), jnp.float32),
        compiler_params=pltpu.CompilerParams(
            dimension_semantics=("arbitrary",)),
    )(s, lp, w_msg)


def _final_tc(f_nuc, nodesum_p, w_a, w_b, lens, seg):
    n, fd = f_nuc.shape
    h = w_a.shape[0]
    b = n // seg

    def body(fn_ref, ns_ref, wa_ref, wb_ref, len_ref, emb_ref, seg_ref):
        e = jnp.maximum(
            _dot_t(fn_ref[...], wa_ref[...]) + _dot_t(ns_ref[...], wb_ref[...]),
            0.0)
        emb_ref[...] = e
        seg_ref[...] = (jnp.sum(e, axis=0) / len_ref[0, 0, 0])[None, None, :]

    emb, segm = pl.pallas_call(
        body,
        grid=(b,),
        in_specs=[
            pl.BlockSpec((seg, fd), lambda i: (i, 0)),
            pl.BlockSpec((seg, h), lambda i: (i, 0)),
            pl.BlockSpec((h, fd), lambda i: (0, 0)),
            pl.BlockSpec((h, h), lambda i: (0, 0)),
            pl.BlockSpec((1, 1, 1), lambda i: (i, 0, 0)),
        ],
        out_specs=[
            pl.BlockSpec((seg, h), lambda i: (i, 0)),
            pl.BlockSpec((1, 1, h), lambda i: (i, 0, 0)),
        ],
        out_shape=[
            jax.ShapeDtypeStruct((n, h), jnp.float32),
            jax.ShapeDtypeStruct((b, 1, h), jnp.float32),
        ],
        compiler_params=pltpu.CompilerParams(
            dimension_semantics=("arbitrary",)),
    )(f_nuc, nodesum_p, w_a, w_b, lens)
    return emb, segm.reshape(b, h)


def _pad_to(x, rows):
    return jnp.pad(x, ((0, rows - x.shape[0]),) + ((0, 0),) * (x.ndim - 1))


def kernel(f_nuc, f_bond, node_graph, message_graph, scope, W_local, W_msg, W_node):
    m = message_graph.shape[0]
    n = node_graph.shape[0]
    b = scope.shape[0]
    seg = n // b
    fd = f_nuc.shape[1]
    h = W_local.shape[0]

    tile = _CHUNK * _NW
    mp = -(-m // tile) * tile
    np_ = -(-n // tile) * tile

    mg = message_graph.astype(jnp.int32)
    ng = node_graph.astype(jnp.int32)
    mg0 = _pad_to(mg[:, 0], mp)
    mg1 = _pad_to(mg[:, 1], mp)
    mg2 = _pad_to(mg[:, 2], mp)
    ng0 = _pad_to(ng[:, 0], np_)
    ng1 = _pad_to(ng[:, 1], np_)
    ng2 = _pad_to(ng[:, 2], np_)
    fb_p = _pad_to(f_bond, mp)

    rb = 2048
    lp, msgs = _local_tc(fb_p, W_local, rb)
    for _ in range(1, _DEPTH):
        s = _gather_sum_sc(msgs, mg0, mg1, mg2)
        msgs = _msg_update_tc(s, lp, W_msg, rb)

    nodesum = _gather_sum_sc(msgs, ng0, ng1, ng2)

    w_a = W_node[:, :fd]
    w_b = W_node[:, fd:]
    lens = scope[:, 1].astype(jnp.float32).reshape(b, 1, 1)
    emb, batch_vec = _final_tc(f_nuc, nodesum, w_a, w_b, lens, seg)
    return (emb, batch_vec)


# speedup vs baseline: 3.2059x; 1.1181x over previous
import jax
import jax.numpy as jnp
from jax import lax
from jax.experimental import pallas as pl
from jax.experimental.pallas import tpu as pltpu
from jax.experimental.pallas import tpu_sc as plsc

_DEPTH = 5
_NC = 2
_NS = 16
_NW = _NC * _NS
_CHUNK = 128


def _gather_sum_sc(table, i0, i1, i2):
    rp = i0.shape[0]
    h = table.shape[1]
    hv = h // 16
    n_chunks = rp // _CHUNK
    cpw = n_chunks // _NW
    mesh = plsc.VectorSubcoreMesh(core_axis_name="c", subcore_axis_name="s")

    def body(i0_hbm, i1_hbm, i2_hbm, table_hbm, out_hbm, i0v, i1v, i2v,
             bufs, sems):
        wid = lax.axis_index("s") * _NC + lax.axis_index("c")
        base = wid * (cpw * _CHUNK)
        pltpu.sync_copy(i0_hbm.at[pl.ds(base, cpw * _CHUNK)], i0v)
        pltpu.sync_copy(i1_hbm.at[pl.ds(base, cpw * _CHUNK)], i1v)
        pltpu.sync_copy(i2_hbm.at[pl.ds(base, cpw * _CHUNK)], i2v)
        idxv = (i0v, i1v, i2v)

        def fire(j, slot):
            return [
                pltpu.async_copy(
                    table_hbm.at[idxv[k].at[pl.ds(j * _CHUNK, _CHUNK)]],
                    bufs.at[slot, k], sems.at[slot])
                for k in range(3)
            ]

        copies = [None, None]
        copies[0] = fire(0, 0)
        for j in range(cpw):
            slot = j % 2
            if j + 1 < cpw:
                copies[1 - slot] = fire(j + 1, 1 - slot)
            for c in copies[slot]:
                c.wait()

            def row_body(r, acc):
                for v in range(hv):
                    sl = pl.ds(v * 16, 16)
                    bufs[slot, 0, r, sl] = (bufs[slot, 0, r, sl]
                                            + bufs[slot, 1, r, sl]
                                            + bufs[slot, 2, r, sl])
                return acc

            lax.fori_loop(0, _CHUNK, row_body, 0)
            pltpu.sync_copy(bufs.at[slot, 0],
                            out_hbm.at[pl.ds(base + j * _CHUNK, _CHUNK)])

    k = pl.kernel(
        body,
        out_type=jax.ShapeDtypeStruct((rp, h), jnp.float32),
        mesh=mesh,
        scratch_types=[
            pltpu.VMEM((cpw * _CHUNK,), jnp.int32),
            pltpu.VMEM((cpw * _CHUNK,), jnp.int32),
            pltpu.VMEM((cpw * _CHUNK,), jnp.int32),
            pltpu.VMEM((2, 3, _CHUNK, h), jnp.float32),
            pltpu.SemaphoreType.DMA((2,)),
        ],
    )
    return k(i0, i1, i2, table)


def _dot_t(a, b):
    return lax.dot_general(a, b, (((1,), (1,)), ((), ())),
                           preferred_element_type=jnp.float32)


def _local_tc(f_bond_p, w_local, rb):
    mp, fd = f_bond_p.shape
    h = w_local.shape[0]

    def body(fb_ref, wl_ref, lp_ref, msg_ref):
        lp = _dot_t(fb_ref[...], wl_ref[...])
        lp_ref[...] = lp
        msg_ref[...] = jnp.maximum(lp, 0.0)

    return pl.pallas_call(
        body,
        grid=(mp // rb,),
        in_specs=[
            pl.BlockSpec((rb, fd), lambda i: (i, 0)),
            pl.BlockSpec((h, fd), lambda i: (0, 0)),
        ],
        out_specs=[
            pl.BlockSpec((rb, h), lambda i: (i, 0)),
            pl.BlockSpec((rb, h), lambda i: (i, 0)),
        ],
        out_shape=[
            jax.ShapeDtypeStruct((mp, h), jnp.float32),
            jax.ShapeDtypeStruct((mp, h), jnp.float32),
        ],
        compiler_params=pltpu.CompilerParams(
            dimension_semantics=("arbitrary",)),
    )(f_bond_p, w_local)


def _msg_update_tc(s, lp, w_msg, rb):
    mp, h = s.shape

    def body(s_ref, lp_ref, w_ref, out_ref):
        out_ref[...] = jnp.maximum(
            lp_ref[...] + _dot_t(s_ref[...], w_ref[...]), 0.0)

    return pl.pallas_call(
        body,
        grid=(mp // rb,),
        in_specs=[
            pl.BlockSpec((rb, h), lambda i: (i, 0)),
            pl.BlockSpec((rb, h), lambda i: (i, 0)),
            pl.BlockSpec((h, h), lambda i: (0, 0)),
        ],
        out_specs=pl.BlockSpec((rb, h), lambda i: (i, 0)),
        out_shape=jax.ShapeDtypeStruct((mp, h), jnp.float32),
        compiler_params=pltpu.CompilerParams(
            dimension_semantics=("arbitrary",)),
    )(s, lp, w_msg)


def _final_tc(f_nuc, nodesum_p, w_a, w_b, lens, seg):
    n, fd = f_nuc.shape
    h = w_a.shape[0]
    b = n // seg

    def body(fn_ref, ns_ref, wa_ref, wb_ref, len_ref, emb_ref, seg_ref):
        e = jnp.maximum(
            _dot_t(fn_ref[...], wa_ref[...]) + _dot_t(ns_ref[...], wb_ref[...]),
            0.0)
        emb_ref[...] = e
        seg_ref[...] = (jnp.sum(e, axis=0) / len_ref[0, 0, 0])[None, None, :]

    emb, segm = pl.pallas_call(
        body,
        grid=(b,),
        in_specs=[
            pl.BlockSpec((seg, fd), lambda i: (i, 0)),
            pl.BlockSpec((seg, h), lambda i: (i, 0)),
            pl.BlockSpec((h, fd), lambda i: (0, 0)),
            pl.BlockSpec((h, h), lambda i: (0, 0)),
            pl.BlockSpec((1, 1, 1), lambda i: (i, 0, 0)),
        ],
        out_specs=[
            pl.BlockSpec((seg, h), lambda i: (i, 0)),
            pl.BlockSpec((1, 1, h), lambda i: (i, 0, 0)),
        ],
        out_shape=[
            jax.ShapeDtypeStruct((n, h), jnp.float32),
            jax.ShapeDtypeStruct((b, 1, h), jnp.float32),
        ],
        compiler_params=pltpu.CompilerParams(
            dimension_semantics=("arbitrary",)),
    )(f_nuc, nodesum_p, w_a, w_b, lens)
    return emb, segm.reshape(b, h)


def _pad_to(x, rows):
    return jnp.pad(x, ((0, rows - x.shape[0]),) + ((0, 0),) * (x.ndim - 1))


def kernel(f_nuc, f_bond, node_graph, message_graph, scope, W_local, W_msg, W_node):
    m = message_graph.shape[0]
    n = node_graph.shape[0]
    b = scope.shape[0]
    seg = n // b
    fd = f_nuc.shape[1]
    h = W_local.shape[0]

    tile = _CHUNK * _NW
    mp = -(-m // tile) * tile
    np_ = -(-n // tile) * tile

    mg = message_graph.astype(jnp.int32)
    ng = node_graph.astype(jnp.int32)
    mg0 = _pad_to(mg[:, 0], mp)
    mg1 = _pad_to(mg[:, 1], mp)
    mg2 = _pad_to(mg[:, 2], mp)
    ng0 = _pad_to(ng[:, 0], np_)
    ng1 = _pad_to(ng[:, 1], np_)
    ng2 = _pad_to(ng[:, 2], np_)
    fb_p = _pad_to(f_bond, mp)

    rb = 2048
    lp, msgs = _local_tc(fb_p, W_local, rb)
    for _ in range(1, _DEPTH):
        s = _gather_sum_sc(msgs, mg0, mg1, mg2)
        msgs = _msg_update_tc(s, lp, W_msg, rb)

    nodesum = _gather_sum_sc(msgs, ng0, ng1, ng2)

    w_a = W_node[:, :fd]
    w_b = W_node[:, fd:]
    lens = scope[:, 1].astype(jnp.float32).reshape(b, 1, 1)
    emb, batch_vec = _final_tc(f_nuc, nodesum, w_a, w_b, lens, seg)
    return (emb, batch_vec)
